# Initial kernel scaffold; baseline (speedup 1.0000x reference)
import jax
import jax.numpy as jnp
from jax.experimental import pallas as pl


def kernel(full_acts, part_acts, eps, enc0_w, enc0_b, enc1_w, enc1_b, enc2_w, enc2_b, enc3_w, enc3_b, prior0_w, prior0_b, prior1_w, prior1_b, prior2_w, prior2_b, prior3_w, prior3_b, res0_w, res0_b, res1_w, res1_b, res2_w, res2_b, res3_w, res3_b, zp_w, zp_b):
    raise NotImplementedError("write your pallas kernel here")


# speedup vs baseline: 1.1946x; 1.1946x over previous
import functools

import jax
import jax.numpy as jnp
from jax.experimental import pallas as pl
from jax.experimental.pallas import tpu as pltpu

_SQRT1_2 = 0.7071067811865476


def _gelu(x):
    return 0.5 * x * (1.0 + jax.lax.erf(x * _SQRT1_2))


def _kl_term(mu1, mu2, ls1, ls2):
    return -0.5 + ls2 - ls1 + 0.5 * (
        jnp.exp(2.0 * (ls1 - ls2)) + (mu1 - mu2) ** 2 * jnp.exp(-2.0 * ls2))


def _vdvae_kernel(full_ref, part_ref, eps_ref,
                  ew0, eb0, ew1, eb1, ew2, eb2, ew3, eb3,
                  pw0, pb0, pw1, pb1, pw2, pb2, pw3, pb3,
                  zw, zb, rw_ref, rb_ref,
                  x_ref, small_ref, *, zd):
    full = full_ref[0]
    fvec = jnp.mean(full, axis=1, keepdims=True)
    pvec = jnp.mean(part_ref[0], axis=1, keepdims=True)

    def mlp(v, layers):
        for w_ref, b_ref in layers:
            v = jnp.dot(w_ref[...], _gelu(v),
                        preferred_element_type=jnp.float32) + b_ref[...]
        return v

    enc = mlp(fvec, ((ew0, eb0), (ew1, eb1), (ew2, eb2), (ew3, eb3)))
    pri = mlp(pvec, ((pw0, pb0), (pw1, pb1), (pw2, pb2), (pw3, pb3)))

    qm, qv = enc[0:zd], enc[zd:2 * zd]
    pm, pv = pri[0:zd], pri[zd:2 * zd]
    xpp = pri[2 * zd:]
    eps = eps_ref[0]

    z = jnp.exp(qv) * eps + qm
    xs = xpp + jnp.dot(zw[...], z,
                       preferred_element_type=jnp.float32) + zb[...]

    kl = _kl_term(qm, pm, qv, pv)
    klq = _kl_term(qm, 0.0, qv, 0.0)
    klp = _kl_term(pm, 0.0, pv, 0.0)
    small_ref[0] = jnp.concatenate([z, kl, klq, klp], axis=0)

    xin = full + xs
    h = xin
    for i in range(4):
        g = _gelu(h).astype(jnp.bfloat16)
        h = jnp.dot(rw_ref[i], g,
                    preferred_element_type=jnp.float32) + rb_ref[i]
    x_ref[0] = xin + h


def kernel(full_acts, part_acts, eps,
           enc0_w, enc0_b, enc1_w, enc1_b, enc2_w, enc2_b, enc3_w, enc3_b,
           prior0_w, prior0_b, prior1_w, prior1_b, prior2_w, prior2_b,
           prior3_w, prior3_b,
           res0_w, res0_b, res1_w, res1_b, res2_w, res2_b, res3_w, res3_b,
           zp_w, zp_b):
    B, C, H, W = full_acts.shape
    HW = H * W
    zd = eps.shape[1]

    full = full_acts.reshape(B, C, HW)
    part = part_acts.reshape(B, C, HW)
    eps3 = eps[:, :, None]

    ewt = [w.T for w in (enc0_w, enc1_w, enc2_w, enc3_w)]
    ebc = [b.T for b in (enc0_b, enc1_b, enc2_b, enc3_b)]
    pwt = [w.T for w in (prior0_w, prior1_w, prior2_w, prior3_w)]
    pbc = [b.T for b in (prior0_b, prior1_b, prior2_b, prior3_b)]
    zwt, zbc = zp_w.T, zp_b.T
    rwt = jnp.stack([res0_w.T, res1_w.T, res2_w.T, res3_w.T]).astype(jnp.bfloat16)
    rbc = jnp.stack([res0_b.T, res1_b.T, res2_b.T, res3_b.T])

    whole = lambda a: pl.BlockSpec(a.shape, lambda b: (0,) * a.ndim)
    x_flat, small = pl.pallas_call(
        functools.partial(_vdvae_kernel, zd=zd),
        grid=(B,),
        in_specs=[pl.BlockSpec((1, C, HW), lambda b: (b, 0, 0)),
                  pl.BlockSpec((1, C, HW), lambda b: (b, 0, 0)),
                  pl.BlockSpec((1, zd, 1), lambda b: (b, 0, 0)),
                  whole(ewt[0]), whole(ebc[0]), whole(ewt[1]), whole(ebc[1]),
                  whole(ewt[2]), whole(ebc[2]), whole(ewt[3]), whole(ebc[3]),
                  whole(pwt[0]), whole(pbc[0]), whole(pwt[1]), whole(pbc[1]),
                  whole(pwt[2]), whole(pbc[2]), whole(pwt[3]), whole(pbc[3]),
                  whole(zwt), whole(zbc), whole(rwt), whole(rbc)],
        out_specs=(pl.BlockSpec((1, C, HW), lambda b: (b, 0, 0)),
                   pl.BlockSpec((1, 4 * zd, 1), lambda b: (b, 0, 0))),
        out_shape=(jax.ShapeDtypeStruct((B, C, HW), jnp.float32),
                   jax.ShapeDtypeStruct((B, 4 * zd, 1), jnp.float32)),
        compiler_params=pltpu.CompilerParams(
            dimension_semantics=("parallel",),
            vmem_limit_bytes=48 * 1024 * 1024),
    )(full, part, eps3,
      ewt[0], ebc[0], ewt[1], ebc[1], ewt[2], ebc[2], ewt[3], ebc[3],
      pwt[0], pbc[0], pwt[1], pbc[1], pwt[2], pbc[2], pwt[3], pbc[3],
      zwt, zbc, rwt, rbc)

    x = x_flat.reshape(B, C, H, W)
    sm = small[:, :, 0]
    z, kl = sm[:, :zd], sm[:, zd:2 * zd]
    klq, klp = sm[:, 2 * zd:3 * zd], sm[:, 3 * zd:]
    to4 = lambda v: v[:, :, None, None]
    return to4(z), x, to4(kl), to4(klp), to4(klq)
